# baseline (speedup 1.0000x reference)
import functools

import jax
import jax.numpy as jnp
from jax import lax
from jax.experimental import pallas as pl
from jax.experimental.pallas import tpu as pltpu
from jax.experimental.pallas import tpu_sc as plsc

_N = 10000
_E = 320000
_D = 128
_DE = 16
_NG = 64

_NC = 2
_NS = 16
_NW = _NC * _NS
_N_PAD = 10112
_CH = 128
_RCH = 2592
_E_PAD = _RCH * _CH
_CPW = _RCH // _NW
_NB = 3
_ROWS_PER_TILE = _N_PAD // _NS

_f32 = jnp.float32


def _enc_body(attr_ref, we_ref, be_ref, e0_ref, e1_ref, e2_ref):
    a = attr_ref[...]
    for l, ref in enumerate((e0_ref, e1_ref, e2_ref)):
        ref[...] = (
            jnp.dot(a, we_ref[l], preferred_element_type=_f32) + be_ref[l]
        )


def _encode(attr_p, we, be):
    blk = 1024
    grid = _E_PAD // blk
    return pl.pallas_call(
        _enc_body,
        grid=(grid,),
        in_specs=[
            pl.BlockSpec((blk, _DE), lambda i: (i, 0)),
            pl.BlockSpec((3, _DE, _D), lambda i: (0, 0, 0)),
            pl.BlockSpec((3, _D), lambda i: (0, 0)),
        ],
        out_specs=[pl.BlockSpec((blk, _D), lambda i: (i, 0))] * 3,
        out_shape=[jax.ShapeDtypeStruct((_E_PAD, _D), _f32)] * 3,
    )(attr_p, we, be)


def _sc_edge_body(h_hbm, e_hbm, sd_hbm, zero_hbm, a0_hbm, a1_hbm, *scr):
    sd_v = scr[0:_NB]
    e_v = scr[_NB:2 * _NB]
    agg_sh = scr[2 * _NB]
    esem = scr[2 * _NB + 1:2 * _NB + 1 + _NB]
    gsem = scr[2 * _NB + 1 + _NB:2 * _NB + 1 + 2 * _NB]
    ssem = scr[2 * _NB + 1 + 2 * _NB:2 * _NB + 1 + 3 * _NB]

    cid = lax.axis_index("c")
    sid = lax.axis_index("s")
    wid = sid * _NC + cid
    base = wid * _CPW
    tile_rows = pl.ds(sid * _ROWS_PER_TILE, _ROWS_PER_TILE)

    pltpu.sync_copy(zero_hbm.at[tile_rows, :], agg_sh.at[tile_rows, :])
    plsc.subcore_barrier()

    for b in range(2):
        pltpu.async_copy(sd_hbm.at[base + b], sd_v[b], esem[b])
        pltpu.async_copy(e_hbm.at[base + b], e_v[b], esem[b])
    pltpu.make_async_copy(sd_hbm.at[base], sd_v[0], esem[0]).wait()
    pltpu.make_async_copy(e_hbm.at[base], e_v[0], esem[0]).wait()
    pltpu.async_copy(h_hbm.at[sd_v[0].at[0]], e_v[0], gsem[0], add=True)

    def step(i, carry):
        for b in range(_NB):
            r = _NB * i + b
            b1 = (b + 1) % _NB
            pb = (b + 2) % _NB

            @pl.when(r + 1 < _CPW)
            def _():
                pltpu.make_async_copy(
                    sd_hbm.at[base + r + 1], sd_v[b1], esem[b1]).wait()
                pltpu.make_async_copy(
                    e_hbm.at[base + r + 1], e_v[b1], esem[b1]).wait()
                pltpu.async_copy(
                    h_hbm.at[sd_v[b1].at[0]], e_v[b1], gsem[b1], add=True)

            pltpu.make_async_copy(
                h_hbm.at[sd_v[b].at[0]], e_v[b], gsem[b]).wait()

            def relu_row(j, c2):
                for dd in range(8):
                    sl = pl.ds(dd * 16, 16)
                    e_v[b][j, sl] = jnp.maximum(e_v[b][j, sl], 0.0)
                return c2

            lax.fori_loop(0, _CH, relu_row, 0, unroll=2)

            pltpu.async_copy(e_v[b], agg_sh.at[sd_v[b].at[1]], ssem[b],
                             add=True)

            @pl.when(r + 2 < _CPW)
            def _():
                @pl.when(r >= _NB - 2)
                def _():
                    pltpu.make_async_copy(
                        e_v[pb], agg_sh.at[sd_v[pb].at[1]], ssem[pb]).wait()
                pltpu.async_copy(sd_hbm.at[base + r + 2], sd_v[pb], esem[pb])
                pltpu.async_copy(e_hbm.at[base + r + 2], e_v[pb], esem[pb])
        return carry

    lax.fori_loop(0, _CPW // _NB, step, 0)
    for rr in range(_CPW - _NB, _CPW):
        b = rr % _NB
        pltpu.make_async_copy(e_v[b], agg_sh.at[sd_v[b].at[1]], ssem[b]).wait()
    plsc.subcore_barrier()

    @pl.when(cid == 0)
    def _():
        pltpu.sync_copy(agg_sh.at[tile_rows, :], a0_hbm.at[tile_rows, :])

    @pl.when(cid == 1)
    def _():
        pltpu.sync_copy(agg_sh.at[tile_rows, :], a1_hbm.at[tile_rows, :])


_edge_pass = pl.kernel(
    _sc_edge_body,
    out_type=(
        jax.ShapeDtypeStruct((_N_PAD, _D), _f32),
        jax.ShapeDtypeStruct((_N_PAD, _D), _f32),
    ),
    mesh=plsc.VectorSubcoreMesh(core_axis_name="c", subcore_axis_name="s"),
    scratch_types=(
        [pltpu.VMEM((2, _CH), jnp.int32) for _ in range(_NB)]
        + [pltpu.VMEM((_CH, _D), _f32) for _ in range(_NB)]
        + [pltpu.VMEM_SHARED((_N_PAD, _D), _f32)]
        + [pltpu.SemaphoreType.DMA for _ in range(3 * _NB)]
    ),
)


def _mlp_body(eps_ref, h_ref, a0_ref, a1_ref, w1_ref, b1_ref, w2_ref, b2_ref,
              o_ref):
    z = h_ref[...] * (1.0 + eps_ref[0, 0]) + a0_ref[...] + a1_ref[...]
    t = jnp.maximum(
        jnp.dot(z, w1_ref[...], preferred_element_type=_f32) + b1_ref[...],
        0.0,
    )
    o_ref[...] = jnp.dot(t, w2_ref[...], preferred_element_type=_f32) + b2_ref[...]


def _mlp(h, a0, a1, p):
    blk = 1264
    grid = _N_PAD // blk
    return pl.pallas_call(
        _mlp_body,
        grid=(grid,),
        in_specs=[
            pl.BlockSpec(memory_space=pltpu.SMEM),
            pl.BlockSpec((blk, _D), lambda i: (i, 0)),
            pl.BlockSpec((blk, _D), lambda i: (i, 0)),
            pl.BlockSpec((blk, _D), lambda i: (i, 0)),
            pl.BlockSpec((_D, 2 * _D), lambda i: (0, 0)),
            pl.BlockSpec((1, 2 * _D), lambda i: (0, 0)),
            pl.BlockSpec((2 * _D, _D), lambda i: (0, 0)),
            pl.BlockSpec((1, _D), lambda i: (0, 0)),
        ],
        out_specs=pl.BlockSpec((blk, _D), lambda i: (i, 0)),
        out_shape=jax.ShapeDtypeStruct((_N_PAD, _D), _f32),
    )(
        p["eps"].reshape(1, 1),
        h,
        a0,
        a1,
        p["W1"],
        p["b1"].reshape(1, 2 * _D),
        p["W2"],
        p["b2"].reshape(1, _D),
    )


def _pool_body(batch_ref, h_ref, wl1_ref, bl1_ref, wl2_ref, bl2_ref, o_ref):
    b = batch_ref[...]
    seg = lax.broadcasted_iota(jnp.int32, (_N_PAD, _NG), 1)
    mask = (b == seg).astype(_f32)
    g = lax.dot_general(
        mask, h_ref[...], (((0,), (0,)), ((), ())),
        preferred_element_type=_f32,
    )
    t = jnp.maximum(
        jnp.dot(g, wl1_ref[...], preferred_element_type=_f32) + bl1_ref[...],
        0.0,
    )
    o_ref[...] = jnp.dot(t, wl2_ref[...], preferred_element_type=_f32) + bl2_ref[...]


def _pool(h, batch_p, wl1, bl1, wl2, bl2):
    return pl.pallas_call(
        _pool_body,
        out_shape=jax.ShapeDtypeStruct((_NG, 1), _f32),
    )(batch_p, h, wl1, bl1.reshape(1, 64), wl2, bl2.reshape(1, 1))


@jax.jit
def kernel(x, edge_index, edge_attr, batch, params):
    src = edge_index[0]
    dst = edge_index[1]
    pad_e = _E_PAD - _E
    src_p = jnp.concatenate([src, jnp.zeros((pad_e,), jnp.int32)]).reshape(
        _RCH, _CH)
    dst_p = jnp.concatenate([dst, jnp.full((pad_e,), _N, jnp.int32)]).reshape(
        _RCH, _CH)
    sd_p = jnp.stack([src_p, dst_p], axis=1)
    attr_p = jnp.concatenate(
        [edge_attr, jnp.zeros((pad_e, _DE), _f32)], axis=0)
    h = jnp.concatenate([x, jnp.zeros((_N_PAD - _N, _D), _f32)], axis=0)
    zeros_nd = jnp.zeros((_N_PAD, _D), _f32)
    batch_p = jnp.concatenate(
        [batch, jnp.full((_N_PAD - _N,), _NG, jnp.int32)]).reshape(_N_PAD, 1)

    we = jnp.stack([p["We"] for p in params["convs"]])
    be = jnp.stack([p["be"] for p in params["convs"]])
    es = _encode(attr_p, we, be)

    for l, p in enumerate(params["convs"]):
        e3 = es[l].reshape(_RCH, _CH, _D)
        a0, a1 = _edge_pass(h, e3, sd_p, zeros_nd)
        h = _mlp(h, a0, a1, p)

    return _pool(h, batch_p, params["Wl1"], params["bl1"], params["Wl2"],
                 params["bl2"])


# speedup vs baseline: 1.1728x; 1.1728x over previous
import jax
import jax.numpy as jnp
from jax import lax
from jax.experimental import pallas as pl
from jax.experimental.pallas import tpu as pltpu
from jax.experimental.pallas import tpu_sc as plsc

_N = 10000
_E = 320000
_D = 128
_DE = 16
_NG = 64

_NC = 2
_NS = 16
_NW = _NC * _NS
_N_PAD = 10240
_CH = 64
_RCH = 5120
_E_PAD = _RCH * _CH
_CPW = _RCH // _NW
_NB = 4
_ROWS_PER_TILE = _N_PAD // _NS
_DP = _D // 2

_f32 = jnp.float32
_bf16 = jnp.bfloat16


def _pack_cols(o):
    outs = []
    for c in range(4):
        a = o[:, c * 32:c * 32 + 16]
        b = o[:, c * 32 + 16:c * 32 + 32]
        ai = lax.bitcast_convert_type(a.astype(_bf16).astype(_f32),
                                      jnp.int32)
        bi = lax.bitcast_convert_type(b.astype(_bf16).astype(_f32),
                                      jnp.int32)
        w = lax.bitwise_or(lax.shift_right_logical(ai, 16),
                           lax.bitwise_and(bi, jnp.int32(-65536)))
        outs.append(lax.bitcast_convert_type(w, _f32))
    return jnp.concatenate(outs, axis=1)


def _enc_body(attr_ref, we_ref, be_ref, e0_ref, e1_ref, e2_ref):
    a = attr_ref[...]
    for l, ref in enumerate((e0_ref, e1_ref, e2_ref)):
        e = jnp.dot(a, we_ref[l], preferred_element_type=_f32) + be_ref[l]
        ref[...] = _pack_cols(e)


def _encode(attr_p, we, be):
    blk = 1024
    grid = _E_PAD // blk
    return pl.pallas_call(
        _enc_body,
        grid=(grid,),
        in_specs=[
            pl.BlockSpec((blk, _DE), lambda i: (i, 0)),
            pl.BlockSpec((3, _DE, _D), lambda i: (0, 0, 0)),
            pl.BlockSpec((3, _D), lambda i: (0, 0)),
        ],
        out_specs=[pl.BlockSpec((blk, _DP), lambda i: (i, 0))] * 3,
        out_shape=[jax.ShapeDtypeStruct((_E_PAD, _DP), _f32)] * 3,
    )(attr_p, we, be)


def _cast_body(h_ref, o_ref):
    o_ref[...] = _pack_cols(h_ref[...])


def _cast_packed(h):
    blk = 1280
    return pl.pallas_call(
        _cast_body,
        grid=(_N_PAD // blk,),
        in_specs=[pl.BlockSpec((blk, _D), lambda i: (i, 0))],
        out_specs=pl.BlockSpec((blk, _DP), lambda i: (i, 0)),
        out_shape=jax.ShapeDtypeStruct((_N_PAD, _DP), _f32),
    )(h)


def _sc_edge_body(h_hbm, e_hbm, sd_hbm, zero_hbm, a0_hbm, a1_hbm, *scr):
    sd_v = scr[0:_NB]
    e_v = scr[_NB:2 * _NB]
    hb_v = scr[2 * _NB:2 * _NB + 2]
    msg_v = scr[2 * _NB + 2:2 * _NB + 4]
    agg_sh = scr[2 * _NB + 4]
    o = 2 * _NB + 5
    esem = scr[o:o + _NB]
    gsem = scr[o + _NB:o + _NB + 2]
    msem = scr[o + _NB + 2:o + _NB + 4]

    cid = lax.axis_index("c")
    sid = lax.axis_index("s")
    wid = sid * _NC + cid
    base = wid * _CPW
    tile_rows = pl.ds(sid * _ROWS_PER_TILE, _ROWS_PER_TILE)

    pltpu.sync_copy(zero_hbm.at[tile_rows, :], agg_sh.at[tile_rows, :])
    plsc.subcore_barrier()

    for b in range(2):
        pltpu.async_copy(sd_hbm.at[base + b], sd_v[b], esem[b])
        pltpu.async_copy(e_hbm.at[base + b], e_v[b], esem[b])
    pltpu.make_async_copy(sd_hbm.at[base], sd_v[0], esem[0]).wait()
    pltpu.make_async_copy(e_hbm.at[base], e_v[0], esem[0]).wait()
    pltpu.async_copy(h_hbm.at[sd_v[0].at[0]], hb_v[0], gsem[0])

    def step(i, carry):
        for b in range(_NB):
            r = _NB * i + b
            b1 = (b + 1) % _NB
            pb = (b + 2) % _NB
            par = b % 2
            npar = (b + 1) % 2

            @pl.when(r + 1 < _CPW)
            def _():
                pltpu.make_async_copy(
                    sd_hbm.at[base + r + 1], sd_v[b1], esem[b1]).wait()
                pltpu.make_async_copy(
                    e_hbm.at[base + r + 1], e_v[b1], esem[b1]).wait()
                pltpu.async_copy(
                    h_hbm.at[sd_v[b1].at[0]], hb_v[npar], gsem[npar])

            pltpu.make_async_copy(
                h_hbm.at[sd_v[b].at[0]], hb_v[par], gsem[par]).wait()

            @pl.when(r >= 2)
            def _():
                pltpu.make_async_copy(
                    msg_v[par], agg_sh.at[sd_v[pb].at[1]], msem[par]).wait()

            def relu_row(j, c2):
                for c in range(4):
                    sl = pl.ds(c * 16, 16)
                    we = lax.bitcast_convert_type(e_v[b][j, sl], jnp.int32)
                    wh = lax.bitcast_convert_type(hb_v[par][j, sl],
                                                  jnp.int32)
                    lo = (lax.bitcast_convert_type(
                              lax.shift_left(we, 16), _f32)
                          + lax.bitcast_convert_type(
                              lax.shift_left(wh, 16), _f32))
                    hi = (lax.bitcast_convert_type(
                              lax.bitwise_and(we, jnp.int32(-65536)), _f32)
                          + lax.bitcast_convert_type(
                              lax.bitwise_and(wh, jnp.int32(-65536)), _f32))
                    msg_v[par][j, pl.ds(c * 32, 16)] = jnp.maximum(lo, 0.0)
                    msg_v[par][j, pl.ds(c * 32 + 16, 16)] = jnp.maximum(
                        hi, 0.0)
                return c2

            lax.fori_loop(0, _CH, relu_row, 0, unroll=2)

            pltpu.async_copy(msg_v[par], agg_sh.at[sd_v[b].at[1]],
                             msem[par], add=True)

            @pl.when(r + 2 < _CPW)
            def _():
                pltpu.async_copy(sd_hbm.at[base + r + 2], sd_v[pb], esem[pb])
                pltpu.async_copy(e_hbm.at[base + r + 2], e_v[pb], esem[pb])
        return carry

    lax.fori_loop(0, _CPW // _NB, step, 0)
    for rr in range(_CPW - 2, _CPW):
        par = rr % 2
        b = rr % _NB
        pltpu.make_async_copy(
            msg_v[par], agg_sh.at[sd_v[b].at[1]], msem[par]).wait()
    plsc.subcore_barrier()

    @pl.when(cid == 0)
    def _():
        pltpu.sync_copy(agg_sh.at[tile_rows, :], a0_hbm.at[tile_rows, :])

    @pl.when(cid == 1)
    def _():
        pltpu.sync_copy(agg_sh.at[tile_rows, :], a1_hbm.at[tile_rows, :])


_edge_pass = pl.kernel(
    _sc_edge_body,
    out_type=(
        jax.ShapeDtypeStruct((_N_PAD, _D), _f32),
        jax.ShapeDtypeStruct((_N_PAD, _D), _f32),
    ),
    mesh=plsc.VectorSubcoreMesh(core_axis_name="c", subcore_axis_name="s"),
    compiler_params=pltpu.CompilerParams(use_tc_tiling_on_sc=False),
    scratch_types=(
        [pltpu.VMEM((2, _CH), jnp.int32) for _ in range(_NB)]
        + [pltpu.VMEM((_CH, _DP), _f32) for _ in range(_NB)]
        + [pltpu.VMEM((_CH, _DP), _f32) for _ in range(2)]
        + [pltpu.VMEM((_CH, _D), _f32) for _ in range(2)]
        + [pltpu.VMEM_SHARED((_N_PAD, _D), _f32)]
        + [pltpu.SemaphoreType.DMA for _ in range(_NB + 4)]
    ),
)


def _mlp_body(eps_ref, h_ref, a0_ref, a1_ref, w1_ref, b1_ref, w2_ref, b2_ref,
              o_ref, op_ref):
    z = h_ref[...] * (1.0 + eps_ref[0, 0]) + a0_ref[...] + a1_ref[...]
    t = jnp.maximum(
        jnp.dot(z, w1_ref[...], preferred_element_type=_f32) + b1_ref[...],
        0.0,
    )
    o = jnp.dot(t, w2_ref[...], preferred_element_type=_f32) + b2_ref[...]
    o_ref[...] = o
    op_ref[...] = _pack_cols(o)


def _mlp(h, a0, a1, p):
    blk = 1280
    grid = _N_PAD // blk
    return pl.pallas_call(
        _mlp_body,
        grid=(grid,),
        in_specs=[
            pl.BlockSpec(memory_space=pltpu.SMEM),
            pl.BlockSpec((blk, _D), lambda i: (i, 0)),
            pl.BlockSpec((blk, _D), lambda i: (i, 0)),
            pl.BlockSpec((blk, _D), lambda i: (i, 0)),
            pl.BlockSpec((_D, 2 * _D), lambda i: (0, 0)),
            pl.BlockSpec((1, 2 * _D), lambda i: (0, 0)),
            pl.BlockSpec((2 * _D, _D), lambda i: (0, 0)),
            pl.BlockSpec((1, _D), lambda i: (0, 0)),
        ],
        out_specs=[
            pl.BlockSpec((blk, _D), lambda i: (i, 0)),
            pl.BlockSpec((blk, _DP), lambda i: (i, 0)),
        ],
        out_shape=[
            jax.ShapeDtypeStruct((_N_PAD, _D), _f32),
            jax.ShapeDtypeStruct((_N_PAD, _DP), _f32),
        ],
    )(
        p["eps"].reshape(1, 1),
        h,
        a0,
        a1,
        p["W1"],
        p["b1"].reshape(1, 2 * _D),
        p["W2"],
        p["b2"].reshape(1, _D),
    )


def _pool_body(batch_ref, h_ref, wl1_ref, bl1_ref, wl2_ref, bl2_ref, o_ref):
    b = batch_ref[...]
    seg = lax.broadcasted_iota(jnp.int32, (_N_PAD, _NG), 1)
    mask = (b == seg).astype(_f32)
    g = lax.dot_general(
        mask, h_ref[...], (((0,), (0,)), ((), ())),
        preferred_element_type=_f32,
    )
    t = jnp.maximum(
        jnp.dot(g, wl1_ref[...], preferred_element_type=_f32) + bl1_ref[...],
        0.0,
    )
    o_ref[...] = jnp.dot(t, wl2_ref[...], preferred_element_type=_f32) + bl2_ref[...]


def _pool(h, batch_p, wl1, bl1, wl2, bl2):
    return pl.pallas_call(
        _pool_body,
        out_shape=jax.ShapeDtypeStruct((_NG, 1), _f32),
    )(batch_p, h, wl1, bl1.reshape(1, 64), wl2, bl2.reshape(1, 1))


@jax.jit
def kernel(x, edge_index, edge_attr, batch, params):
    src = edge_index[0]
    dst = edge_index[1]
    pad_e = _E_PAD - _E
    src_p = jnp.concatenate([src, jnp.zeros((pad_e,), jnp.int32)]).reshape(
        _RCH, _CH)
    dst_p = jnp.concatenate([dst, jnp.full((pad_e,), _N, jnp.int32)]).reshape(
        _RCH, _CH)
    sd_p = jnp.stack([src_p, dst_p], axis=1)
    attr_p = jnp.concatenate(
        [edge_attr, jnp.zeros((pad_e, _DE), _f32)], axis=0)
    h = jnp.concatenate([x, jnp.zeros((_N_PAD - _N, _D), _f32)], axis=0)
    zeros_nd = jnp.zeros((_N_PAD, _D), _f32)
    batch_p = jnp.concatenate(
        [batch, jnp.full((_N_PAD - _N,), _NG, jnp.int32)]).reshape(_N_PAD, 1)

    we = jnp.stack([p["We"] for p in params["convs"]])
    be = jnp.stack([p["be"] for p in params["convs"]])
    es = _encode(attr_p, we, be)

    hp = _cast_packed(h)
    for l, p in enumerate(params["convs"]):
        e3 = es[l].reshape(_RCH, _CH, _DP)
        a0, a1 = _edge_pass(hp, e3, sd_p, zeros_nd)
        h, hp = _mlp(h, a0, a1, p)

    return _pool(h, batch_p, params["Wl1"], params["bl1"], params["Wl2"],
                 params["bl2"])


# speedup vs baseline: 1.2289x; 1.0478x over previous
import jax
import jax.numpy as jnp
from jax import lax
from jax.experimental import pallas as pl
from jax.experimental.pallas import tpu as pltpu
from jax.experimental.pallas import tpu_sc as plsc

_N = 10000
_E = 320000
_D = 128
_DE = 16
_NG = 64

_NC = 2
_NS = 16
_NW = _NC * _NS
_N_PAD = 10240
_CH = 128
_RCH = 2528
_E_PAD = _RCH * _CH
_CPW = _RCH // _NW
_ROWS_PER_TILE = _N_PAD // _NS

_f32 = jnp.float32


def _enc_body(attr_ref, we_ref, be_ref, e0_ref, e1_ref, e2_ref):
    a = attr_ref[...]
    for l, ref in enumerate((e0_ref, e1_ref, e2_ref)):
        ref[...] = (
            jnp.dot(a, we_ref[l], preferred_element_type=_f32) + be_ref[l]
        )


def _encode(attr_p, we, be):
    blk = 1024
    grid = _E_PAD // blk
    return pl.pallas_call(
        _enc_body,
        grid=(grid,),
        in_specs=[
            pl.BlockSpec((blk, _DE), lambda i: (i, 0)),
            pl.BlockSpec((3, _DE, _D), lambda i: (0, 0, 0)),
            pl.BlockSpec((3, _D), lambda i: (0, 0)),
        ],
        out_specs=[pl.BlockSpec((blk, _D), lambda i: (i, 0))] * 3,
        out_shape=[jax.ShapeDtypeStruct((_E_PAD, _D), _f32)] * 3,
    )(attr_p, we, be)


def _sc_edge_body(h_hbm, e_hbm, sd_hbm, zero_hbm, a0_hbm, a1_hbm,
                  sd_v, e_v, agg_sh, sem):
    cid = lax.axis_index("c")
    sid = lax.axis_index("s")
    wid = sid * _NC + cid
    base = wid * _CPW
    tile_rows = pl.ds(sid * _ROWS_PER_TILE, _ROWS_PER_TILE)

    pltpu.sync_copy(zero_hbm.at[tile_rows, :], agg_sh.at[tile_rows, :])
    plsc.subcore_barrier()

    def chunk(r, carry):
        row = base + r
        pltpu.sync_copy(sd_hbm.at[row], sd_v)
        pltpu.sync_copy(e_hbm.at[row], e_v)
        pltpu.async_copy(h_hbm.at[sd_v.at[0]], e_v, sem, add=True).wait()

        def relu_row(j, c2):
            for dd in range(8):
                sl = pl.ds(dd * 16, 16)
                e_v[j, sl] = jnp.maximum(e_v[j, sl], 0.0)
            return c2

        lax.fori_loop(0, _CH, relu_row, 0, unroll=2)
        pltpu.sync_copy(e_v, agg_sh.at[sd_v.at[1]], add=True)
        return carry

    lax.fori_loop(0, _CPW, chunk, 0)
    plsc.subcore_barrier()

    @pl.when(cid == 0)
    def _():
        pltpu.sync_copy(agg_sh.at[tile_rows, :], a0_hbm.at[tile_rows, :])

    @pl.when(cid == 1)
    def _():
        pltpu.sync_copy(agg_sh.at[tile_rows, :], a1_hbm.at[tile_rows, :])


_edge_pass = pl.kernel(
    _sc_edge_body,
    out_type=(
        jax.ShapeDtypeStruct((_N_PAD, _D), _f32),
        jax.ShapeDtypeStruct((_N_PAD, _D), _f32),
    ),
    mesh=plsc.VectorSubcoreMesh(core_axis_name="c", subcore_axis_name="s"),
    scratch_types=[
        pltpu.VMEM((2, _CH), jnp.int32),
        pltpu.VMEM((_CH, _D), _f32),
        pltpu.VMEM_SHARED((_N_PAD, _D), _f32),
        pltpu.SemaphoreType.DMA,
    ],
)


def _mlp_body(eps_ref, h_ref, a0_ref, a1_ref, w1_ref, b1_ref, w2_ref, b2_ref,
              o_ref):
    z = h_ref[...] * (1.0 + eps_ref[0, 0]) + a0_ref[...] + a1_ref[...]
    t = jnp.maximum(
        jnp.dot(z, w1_ref[...], preferred_element_type=_f32) + b1_ref[...],
        0.0,
    )
    o_ref[...] = jnp.dot(t, w2_ref[...], preferred_element_type=_f32) + b2_ref[...]


def _mlp(h, a0, a1, p):
    blk = 1024
    grid = _N_PAD // blk
    return pl.pallas_call(
        _mlp_body,
        grid=(grid,),
        in_specs=[
            pl.BlockSpec(memory_space=pltpu.SMEM),
            pl.BlockSpec((blk, _D), lambda i: (i, 0)),
            pl.BlockSpec((blk, _D), lambda i: (i, 0)),
            pl.BlockSpec((blk, _D), lambda i: (i, 0)),
            pl.BlockSpec((_D, 2 * _D), lambda i: (0, 0)),
            pl.BlockSpec((1, 2 * _D), lambda i: (0, 0)),
            pl.BlockSpec((2 * _D, _D), lambda i: (0, 0)),
            pl.BlockSpec((1, _D), lambda i: (0, 0)),
        ],
        out_specs=pl.BlockSpec((blk, _D), lambda i: (i, 0)),
        out_shape=jax.ShapeDtypeStruct((_N_PAD, _D), _f32),
    )(
        p["eps"].reshape(1, 1),
        h,
        a0,
        a1,
        p["W1"],
        p["b1"].reshape(1, 2 * _D),
        p["W2"],
        p["b2"].reshape(1, _D),
    )


def _pool_body(batch_ref, h_ref, wl1_ref, bl1_ref, wl2_ref, bl2_ref, o_ref):
    b = batch_ref[...]
    seg = lax.broadcasted_iota(jnp.int32, (_N_PAD, _NG), 1)
    mask = (b == seg).astype(_f32)
    g = lax.dot_general(
        mask, h_ref[...], (((0,), (0,)), ((), ())),
        preferred_element_type=_f32,
    )
    t = jnp.maximum(
        jnp.dot(g, wl1_ref[...], preferred_element_type=_f32) + bl1_ref[...],
        0.0,
    )
    o_ref[...] = jnp.dot(t, wl2_ref[...], preferred_element_type=_f32) + bl2_ref[...]


def _pool(h, batch_p, wl1, bl1, wl2, bl2):
    return pl.pallas_call(
        _pool_body,
        out_shape=jax.ShapeDtypeStruct((_NG, 1), _f32),
    )(batch_p, h, wl1, bl1.reshape(1, 64), wl2, bl2.reshape(1, 1))


@jax.jit
def kernel(x, edge_index, edge_attr, batch, params):
    src = edge_index[0]
    dst = edge_index[1]
    pad_e = _E_PAD - _E
    src_p = jnp.concatenate([src, jnp.zeros((pad_e,), jnp.int32)]).reshape(
        _RCH, _CH)
    dst_p = jnp.concatenate([dst, jnp.full((pad_e,), _N, jnp.int32)]).reshape(
        _RCH, _CH)
    sd_p = jnp.stack([src_p, dst_p], axis=1)
    attr_p = jnp.concatenate(
        [edge_attr, jnp.zeros((pad_e, _DE), _f32)], axis=0)
    h = jnp.concatenate([x, jnp.zeros((_N_PAD - _N, _D), _f32)], axis=0)
    zeros_nd = jnp.zeros((_N_PAD, _D), _f32)
    batch_p = jnp.concatenate(
        [batch, jnp.full((_N_PAD - _N,), _NG, jnp.int32)]).reshape(_N_PAD, 1)

    we = jnp.stack([p["We"] for p in params["convs"]])
    be = jnp.stack([p["be"] for p in params["convs"]])
    es = _encode(attr_p, we, be)

    for l, p in enumerate(params["convs"]):
        e3 = es[l].reshape(_RCH, _CH, _D)
        a0, a1 = _edge_pass(h, e3, sd_p, zeros_nd)
        h = _mlp(h, a0, a1, p)

    return _pool(h, batch_p, params["Wl1"], params["bl1"], params["Wl2"],
                 params["bl2"])
